# baseline (speedup 1.0000x reference)
import functools

import jax
import jax.numpy as jnp
from jax import lax
from jax.experimental import pallas as pl
from jax.experimental.pallas import tpu as pltpu
from jax.experimental.pallas import tpu_sc as plsc

N = 10000
E = 320000
D = 128

NC = 2
NS = 16
NW = NC * NS
CHUNK = 128

ROWS_PER_TILE = 632
ACC_ROWS = NS * ROWS_PER_TILE
CHUNKS_PER_TILE = 80
EDGES_PER_TILE = CHUNKS_PER_TILE * CHUNK
E_PAD = NW * EDGES_PER_TILE
NBUF = 2
TOT_CHUNKS = NW * CHUNKS_PER_TILE

_mesh = plsc.VectorSubcoreMesh(core_axis_name="c", subcore_axis_name="s")


@functools.partial(
    pl.kernel,
    out_type=jax.ShapeDtypeStruct((2 * ACC_ROWS, D), jnp.float32),
    mesh=_mesh,
    scratch_types=[
        pltpu.VMEM_SHARED((ACC_ROWS, D), jnp.float32),
        pltpu.VMEM((CHUNK,), jnp.int32),
        pltpu.VMEM((CHUNK,), jnp.int32),
        pltpu.VMEM((CHUNK, D), jnp.float32),
        pltpu.SemaphoreType.DMA,
    ],
)
def _sc_segsum(h_hbm, src_hbm, dst_hbm, zeros_hbm, out_hbm,
               acc, sidx, didx, rows, sem):
    c = lax.axis_index("c")
    s = lax.axis_index("s")
    w = c * NS + s
    eb = w * EDGES_PER_TILE

    pltpu.sync_copy(zeros_hbm, acc.at[pl.ds(s * ROWS_PER_TILE, ROWS_PER_TILE)])
    plsc.subcore_barrier()

    def body(g, carry):
        base = eb + g * CHUNK
        pltpu.sync_copy(src_hbm.at[pl.ds(base, CHUNK)], sidx)
        pltpu.sync_copy(dst_hbm.at[pl.ds(base, CHUNK)], didx)
        pltpu.async_copy(h_hbm.at[sidx], rows, sem).wait()
        pltpu.sync_copy(rows, acc.at[didx], add=True)
        return carry

    lax.fori_loop(0, CHUNKS_PER_TILE, body, 0)
    plsc.subcore_barrier()

    pltpu.sync_copy(acc.at[pl.ds(s * ROWS_PER_TILE, ROWS_PER_TILE)],
                    out_hbm.at[pl.ds(c * ACC_ROWS + s * ROWS_PER_TILE,
                                     ROWS_PER_TILE)])


@functools.partial(
    pl.kernel,
    out_type=jax.ShapeDtypeStruct((2 * ACC_ROWS, D), jnp.float32),
    mesh=_mesh,
    scratch_types=[
        pltpu.VMEM_SHARED((ACC_ROWS, D), jnp.float32),
        pltpu.VMEM((CHUNKS_PER_TILE, CHUNK), jnp.int32),
        pltpu.VMEM((CHUNK, D), jnp.float32),
        [pltpu.SemaphoreType.DMA] * NBUF,
    ],
)
def _sc_degree(dst_hbm, zeros_hbm, ones_hbm, out_hbm,
               accd, didx, ones_v, ssem):
    c = lax.axis_index("c")
    s = lax.axis_index("s")
    w = c * NS + s

    pltpu.sync_copy(zeros_hbm, accd.at[pl.ds(s * ROWS_PER_TILE, ROWS_PER_TILE)])
    pltpu.sync_copy(dst_hbm.at[pl.ds(w * CHUNKS_PER_TILE, CHUNKS_PER_TILE)],
                    didx)
    pltpu.sync_copy(ones_hbm, ones_v)
    plsc.subcore_barrier()

    def body(i, carry):
        base = i * NBUF
        for j in range(NBUF):
            pltpu.async_copy(ones_v, accd.at[didx.at[base + j]], ssem[j],
                             add=True)
        for j in range(NBUF):
            pltpu.make_async_copy(ones_v, accd.at[didx.at[base + j]],
                                  ssem[j]).wait()
        return carry

    lax.fori_loop(0, CHUNKS_PER_TILE // NBUF, body, 0)
    plsc.subcore_barrier()
    pltpu.sync_copy(accd.at[pl.ds(s * ROWS_PER_TILE, ROWS_PER_TILE)],
                    out_hbm.at[pl.ds(c * ACC_ROWS + s * ROWS_PER_TILE,
                                     ROWS_PER_TILE)])


ROWB = 2000


def _fc0_body(x_ref, w_ref, b_ref, o_ref):
    y = jnp.dot(x_ref[...], w_ref[...], preferred_element_type=jnp.float32)
    o_ref[...] = jnp.maximum(y + b_ref[...], 0.0)


def _tc_fc0(x, w, b):
    return pl.pallas_call(
        _fc0_body,
        grid=(N // ROWB,),
        in_specs=[
            pl.BlockSpec((ROWB, D), lambda i: (i, 0)),
            pl.BlockSpec((D, D), lambda i: (0, 0)),
            pl.BlockSpec((1, D), lambda i: (0, 0)),
        ],
        out_specs=pl.BlockSpec((ROWB, D), lambda i: (i, 0)),
        out_shape=jax.ShapeDtypeStruct((N, D), jnp.float32),
    )(x, w, b.reshape(1, D))


def _layer_body(eps_ref, h_ref, p0_ref, p1_ref, r_ref, w_ref, b_ref, o_ref,
                *, relu):
    agg = (p0_ref[...] + p1_ref[...]) * r_ref[...]
    x = (1.0 + eps_ref[0]) * h_ref[...] + agg
    y = jnp.dot(x, w_ref[...], preferred_element_type=jnp.float32) + b_ref[...]
    o_ref[...] = jnp.maximum(y, 0.0) if relu else y


def _tc_layer(h, p0, p1, rdeg, w, b, eps, relu):
    return pl.pallas_call(
        functools.partial(_layer_body, relu=relu),
        grid=(N // ROWB,),
        in_specs=[
            pl.BlockSpec(memory_space=pltpu.SMEM),
            pl.BlockSpec((ROWB, D), lambda i: (i, 0)),
            pl.BlockSpec((ROWB, D), lambda i: (i, 0)),
            pl.BlockSpec((ROWB, D), lambda i: (i, 0)),
            pl.BlockSpec((ROWB, 1), lambda i: (i, 0)),
            pl.BlockSpec((D, D), lambda i: (0, 0)),
            pl.BlockSpec((1, D), lambda i: (0, 0)),
        ],
        out_specs=pl.BlockSpec((ROWB, D), lambda i: (i, 0)),
        out_shape=jax.ShapeDtypeStruct((N, D), jnp.float32),
    )(eps.reshape(1), h, p0, p1, rdeg, w, b.reshape(1, D))


def kernel(features, edge_index, W0, b0, W1, b1, eps1, W2, b2, eps2, W3, b3, eps3):
    src = edge_index[0]
    dst = edge_index[1]
    pad = E_PAD - E
    src_p = jnp.concatenate([src, jnp.zeros((pad,), jnp.int32)])
    dst_p = jnp.concatenate([dst, jnp.full((pad,), N, jnp.int32)])
    dst_p2 = dst_p.reshape(TOT_CHUNKS, CHUNK)
    zeros_hbm = jnp.zeros((ROWS_PER_TILE, D), jnp.float32)
    ones_hbm = jnp.ones((CHUNK, D), jnp.float32)

    dp = _sc_degree(dst_p2, zeros_hbm, ones_hbm)
    deg = dp[:N, 0] + dp[ACC_ROWS:ACC_ROWS + N, 0]
    rdeg = (1.0 / jnp.maximum(deg, 1.0)).reshape(N, 1)

    h = _tc_fc0(features, W0, b0)
    for w_, b_, eps_, relu in ((W1, b1, eps1, True),
                               (W2, b2, eps2, True),
                               (W3, b3, eps3, False)):
        p = _sc_segsum(h, src_p, dst_p, zeros_hbm)
        h = _tc_layer(h, p[:N], p[ACC_ROWS:ACC_ROWS + N], rdeg,
                      w_, b_, eps_, relu)
    return h


# speedup vs baseline: 2.0777x; 2.0777x over previous
import functools

import jax
import jax.numpy as jnp
from jax import lax
from jax.experimental import pallas as pl
from jax.experimental.pallas import tpu as pltpu
from jax.experimental.pallas import tpu_sc as plsc

N = 10000
E = 320000
D = 128

NC = 2
NS = 16
NW = NC * NS
CHUNK = 128

ROWS_PER_TILE = 632
ACC_ROWS = NS * ROWS_PER_TILE
CHUNKS_PER_TILE = 79
EDGES_PER_TILE = CHUNKS_PER_TILE * CHUNK
E_PAD = NW * EDGES_PER_TILE
NBUF = 2
TOT_CHUNKS = NW * CHUNKS_PER_TILE

_mesh = plsc.VectorSubcoreMesh(core_axis_name="c", subcore_axis_name="s")


@functools.partial(
    pl.kernel,
    out_type=jax.ShapeDtypeStruct((2 * ACC_ROWS, D), jnp.float32),
    mesh=_mesh,
    scratch_types=[
        pltpu.VMEM_SHARED((ACC_ROWS, D), jnp.float32),
        pltpu.VMEM((CHUNK,), jnp.int32),
        pltpu.VMEM((CHUNK,), jnp.int32),
        pltpu.VMEM((CHUNK, D), jnp.float32),
        pltpu.SemaphoreType.DMA,
    ],
)
def _sc_segsum(h_hbm, src_hbm, dst_hbm, zeros_hbm, out_hbm,
               acc, sidx, didx, rows, sem):
    c = lax.axis_index("c")
    s = lax.axis_index("s")
    w = c * NS + s
    eb = w * EDGES_PER_TILE

    pltpu.sync_copy(zeros_hbm, acc.at[pl.ds(s * ROWS_PER_TILE, ROWS_PER_TILE)])
    plsc.subcore_barrier()

    def body(g, carry):
        base = eb + g * CHUNK
        pltpu.sync_copy(src_hbm.at[pl.ds(base, CHUNK)], sidx)
        pltpu.sync_copy(dst_hbm.at[pl.ds(base, CHUNK)], didx)
        pltpu.async_copy(h_hbm.at[sidx], rows, sem).wait()
        pltpu.sync_copy(rows, acc.at[didx], add=True)
        return carry

    lax.fori_loop(0, CHUNKS_PER_TILE, body, 0)
    plsc.subcore_barrier()

    pltpu.sync_copy(acc.at[pl.ds(s * ROWS_PER_TILE, ROWS_PER_TILE)],
                    out_hbm.at[pl.ds(c * ACC_ROWS + s * ROWS_PER_TILE,
                                     ROWS_PER_TILE)])


@functools.partial(
    pl.kernel,
    out_type=jax.ShapeDtypeStruct((2 * ACC_ROWS, D), jnp.float32),
    mesh=_mesh,
    scratch_types=[
        pltpu.VMEM_SHARED((ACC_ROWS, D), jnp.float32),
        [pltpu.VMEM((CHUNK,), jnp.int32)] * NBUF,
        pltpu.VMEM((CHUNK, D), jnp.float32),
        [pltpu.SemaphoreType.DMA] * NBUF,
    ],
)
def _sc_degree(dst_hbm, zeros_hbm, ones_hbm, out_hbm,
               accd, didx, ones_v, ssem):
    c = lax.axis_index("c")
    s = lax.axis_index("s")
    w = c * NS + s
    eb = w * EDGES_PER_TILE

    pltpu.sync_copy(zeros_hbm, accd.at[pl.ds(s * ROWS_PER_TILE, ROWS_PER_TILE)])
    pltpu.sync_copy(ones_hbm, ones_v)
    plsc.subcore_barrier()

    def sfire(g, j):
        pltpu.sync_copy(dst_hbm.at[pl.ds(eb + g * CHUNK, CHUNK)], didx[j])
        pltpu.async_copy(ones_v, accd.at[didx[j]], ssem[j], add=True)

    def sdrain(j):
        pltpu.make_async_copy(ones_v, accd.at[didx[j]], ssem[j]).wait()

    for j in range(NBUF):
        sfire(j, j)

    def body(i, carry):
        for j in range(NBUF):
            sdrain(j)
            sfire(i * NBUF + j, j)
        return carry

    lax.fori_loop(1, CHUNKS_PER_TILE // NBUF, body, 0)
    sdrain(0)
    sfire(CHUNKS_PER_TILE - 1, 0)
    for j in range(NBUF):
        sdrain(j)
    plsc.subcore_barrier()
    pltpu.sync_copy(accd.at[pl.ds(s * ROWS_PER_TILE, ROWS_PER_TILE)],
                    out_hbm.at[pl.ds(c * ACC_ROWS + s * ROWS_PER_TILE,
                                     ROWS_PER_TILE)])


ROWB = 2000


def _fc0_body(x_ref, w_ref, b_ref, o_ref):
    y = jnp.dot(x_ref[...], w_ref[...], preferred_element_type=jnp.float32)
    o_ref[...] = jnp.maximum(y + b_ref[...], 0.0)


def _tc_fc0(x, w, b):
    return pl.pallas_call(
        _fc0_body,
        grid=(N // ROWB,),
        in_specs=[
            pl.BlockSpec((ROWB, D), lambda i: (i, 0)),
            pl.BlockSpec((D, D), lambda i: (0, 0)),
            pl.BlockSpec((1, D), lambda i: (0, 0)),
        ],
        out_specs=pl.BlockSpec((ROWB, D), lambda i: (i, 0)),
        out_shape=jax.ShapeDtypeStruct((N, D), jnp.float32),
    )(x, w, b.reshape(1, D))


def _layer_body(eps_ref, h_ref, p0_ref, p1_ref, r_ref, w_ref, b_ref, o_ref,
                *, relu):
    agg = (p0_ref[...] + p1_ref[...]) * r_ref[...]
    x = (1.0 + eps_ref[0]) * h_ref[...] + agg
    y = jnp.dot(x, w_ref[...], preferred_element_type=jnp.float32) + b_ref[...]
    o_ref[...] = jnp.maximum(y, 0.0) if relu else y


def _tc_layer(h, p0, p1, rdeg, w, b, eps, relu):
    return pl.pallas_call(
        functools.partial(_layer_body, relu=relu),
        grid=(N // ROWB,),
        in_specs=[
            pl.BlockSpec(memory_space=pltpu.SMEM),
            pl.BlockSpec((ROWB, D), lambda i: (i, 0)),
            pl.BlockSpec((ROWB, D), lambda i: (i, 0)),
            pl.BlockSpec((ROWB, D), lambda i: (i, 0)),
            pl.BlockSpec((ROWB, 1), lambda i: (i, 0)),
            pl.BlockSpec((D, D), lambda i: (0, 0)),
            pl.BlockSpec((1, D), lambda i: (0, 0)),
        ],
        out_specs=pl.BlockSpec((ROWB, D), lambda i: (i, 0)),
        out_shape=jax.ShapeDtypeStruct((N, D), jnp.float32),
    )(eps.reshape(1), h, p0, p1, rdeg, w, b.reshape(1, D))


def kernel(features, edge_index, W0, b0, W1, b1, eps1, W2, b2, eps2, W3, b3, eps3):
    src = edge_index[0]
    dst = edge_index[1]
    pad = E_PAD - E
    ar = jnp.arange(pad, dtype=jnp.int32)
    src_p = jnp.concatenate([src, ar % N])
    dst_p = jnp.concatenate([dst, N + ar % (ACC_ROWS - N)])
    zeros_hbm = jnp.zeros((ROWS_PER_TILE, D), jnp.float32)
    ones_hbm = jnp.ones((CHUNK, D), jnp.float32)

    dp = _sc_degree(dst_p, zeros_hbm, ones_hbm)
    deg = dp[:N, 0] + dp[ACC_ROWS:ACC_ROWS + N, 0]
    rdeg = (1.0 / jnp.maximum(deg, 1.0)).reshape(N, 1)

    h = _tc_fc0(features, W0, b0)
    for w_, b_, eps_, relu in ((W1, b1, eps1, True),
                               (W2, b2, eps2, True),
                               (W3, b3, eps3, False)):
        p = _sc_segsum(h, src_p, dst_p, zeros_hbm)
        h = _tc_layer(h, p[:N], p[ACC_ROWS:ACC_ROWS + N], rdeg,
                      w_, b_, eps_, relu)
    return h


# speedup vs baseline: 2.4740x; 1.1907x over previous
import functools

import jax
import jax.numpy as jnp
from jax import lax
from jax.experimental import pallas as pl
from jax.experimental.pallas import tpu as pltpu
from jax.experimental.pallas import tpu_sc as plsc

N = 10000
E = 320000
D = 128

NC = 2
NS = 16
NW = NC * NS
CHUNK = 128

ROWS_PER_TILE = 632
ACC_ROWS = NS * ROWS_PER_TILE
CHUNKS_PER_TILE = 79
EDGES_PER_TILE = CHUNKS_PER_TILE * CHUNK
E_PAD = NW * EDGES_PER_TILE
NBUF = 2
TOT_CHUNKS = NW * CHUNKS_PER_TILE

_mesh = plsc.VectorSubcoreMesh(core_axis_name="c", subcore_axis_name="s")


@functools.partial(
    pl.kernel,
    out_type=jax.ShapeDtypeStruct((2 * ACC_ROWS, D), jnp.float32),
    mesh=_mesh,
    scratch_types=[
        pltpu.VMEM_SHARED((ACC_ROWS, D), jnp.float32),
        [pltpu.VMEM((CHUNK,), jnp.int32)] * NBUF,
        [pltpu.VMEM((CHUNK,), jnp.int32)] * NBUF,
        [pltpu.VMEM((CHUNK, D), jnp.float32)] * NBUF,
        pltpu.SemaphoreType.DMA,
        [pltpu.SemaphoreType.DMA] * NBUF,
    ],
)
def _sc_segsum(h_hbm, src_hbm, dst_hbm, zeros_hbm, out_hbm,
               acc, sidx, didx, rows, gsem, ssem):
    c = lax.axis_index("c")
    s = lax.axis_index("s")
    w = c * NS + s
    eb = w * EDGES_PER_TILE

    pltpu.sync_copy(zeros_hbm, acc.at[pl.ds(s * ROWS_PER_TILE, ROWS_PER_TILE)])
    plsc.subcore_barrier()

    def work(g, j):
        base = eb + g * CHUNK
        pltpu.sync_copy(src_hbm.at[pl.ds(base, CHUNK)], sidx[j])
        pltpu.sync_copy(dst_hbm.at[pl.ds(base, CHUNK)], didx[j])
        pltpu.async_copy(h_hbm.at[sidx[j]], rows[j], gsem).wait()
        pltpu.async_copy(rows[j], acc.at[didx[j]], ssem[j], add=True)

    def sdrain(j):
        pltpu.make_async_copy(rows[j], acc.at[didx[j]], ssem[j]).wait()

    for j in range(NBUF):
        work(j, j)

    def body(i, carry):
        for j in range(NBUF):
            sdrain(j)
            work(i * NBUF + j, j)
        return carry

    lax.fori_loop(1, CHUNKS_PER_TILE // NBUF, body, 0)
    sdrain(0)
    work(CHUNKS_PER_TILE - 1, 0)
    for j in range(NBUF):
        sdrain(j)
    plsc.subcore_barrier()

    pltpu.sync_copy(acc.at[pl.ds(s * ROWS_PER_TILE, ROWS_PER_TILE)],
                    out_hbm.at[pl.ds(c * ACC_ROWS + s * ROWS_PER_TILE,
                                     ROWS_PER_TILE)])


@functools.partial(
    pl.kernel,
    out_type=jax.ShapeDtypeStruct((2 * ACC_ROWS, D), jnp.float32),
    mesh=_mesh,
    scratch_types=[
        pltpu.VMEM_SHARED((ACC_ROWS, D), jnp.float32),
        [pltpu.VMEM((CHUNK,), jnp.int32)] * NBUF,
        pltpu.VMEM((CHUNK, D), jnp.float32),
        [pltpu.SemaphoreType.DMA] * NBUF,
    ],
)
def _sc_degree(dst_hbm, zeros_hbm, ones_hbm, out_hbm,
               accd, didx, ones_v, ssem):
    c = lax.axis_index("c")
    s = lax.axis_index("s")
    w = c * NS + s
    eb = w * EDGES_PER_TILE

    pltpu.sync_copy(zeros_hbm, accd.at[pl.ds(s * ROWS_PER_TILE, ROWS_PER_TILE)])
    pltpu.sync_copy(ones_hbm, ones_v)
    plsc.subcore_barrier()

    def sfire(g, j):
        pltpu.sync_copy(dst_hbm.at[pl.ds(eb + g * CHUNK, CHUNK)], didx[j])
        pltpu.async_copy(ones_v, accd.at[didx[j]], ssem[j], add=True)

    def sdrain(j):
        pltpu.make_async_copy(ones_v, accd.at[didx[j]], ssem[j]).wait()

    for j in range(NBUF):
        sfire(j, j)

    def body(i, carry):
        for j in range(NBUF):
            sdrain(j)
            sfire(i * NBUF + j, j)
        return carry

    lax.fori_loop(1, CHUNKS_PER_TILE // NBUF, body, 0)
    sdrain(0)
    sfire(CHUNKS_PER_TILE - 1, 0)
    for j in range(NBUF):
        sdrain(j)
    plsc.subcore_barrier()
    pltpu.sync_copy(accd.at[pl.ds(s * ROWS_PER_TILE, ROWS_PER_TILE)],
                    out_hbm.at[pl.ds(c * ACC_ROWS + s * ROWS_PER_TILE,
                                     ROWS_PER_TILE)])


ROWB = 2000


def _fc0_body(x_ref, w_ref, b_ref, o_ref):
    y = jnp.dot(x_ref[...], w_ref[...], preferred_element_type=jnp.float32)
    o_ref[...] = jnp.maximum(y + b_ref[...], 0.0)


def _tc_fc0(x, w, b):
    return pl.pallas_call(
        _fc0_body,
        grid=(N // ROWB,),
        in_specs=[
            pl.BlockSpec((ROWB, D), lambda i: (i, 0)),
            pl.BlockSpec((D, D), lambda i: (0, 0)),
            pl.BlockSpec((1, D), lambda i: (0, 0)),
        ],
        out_specs=pl.BlockSpec((ROWB, D), lambda i: (i, 0)),
        out_shape=jax.ShapeDtypeStruct((N, D), jnp.float32),
    )(x, w, b.reshape(1, D))


def _layer_body(eps_ref, h_ref, p0_ref, p1_ref, r_ref, w_ref, b_ref, o_ref,
                *, relu):
    agg = (p0_ref[...] + p1_ref[...]) * r_ref[...]
    x = (1.0 + eps_ref[0]) * h_ref[...] + agg
    y = jnp.dot(x, w_ref[...], preferred_element_type=jnp.float32) + b_ref[...]
    o_ref[...] = jnp.maximum(y, 0.0) if relu else y


def _tc_layer(h, p0, p1, rdeg, w, b, eps, relu):
    return pl.pallas_call(
        functools.partial(_layer_body, relu=relu),
        grid=(N // ROWB,),
        in_specs=[
            pl.BlockSpec(memory_space=pltpu.SMEM),
            pl.BlockSpec((ROWB, D), lambda i: (i, 0)),
            pl.BlockSpec((ROWB, D), lambda i: (i, 0)),
            pl.BlockSpec((ROWB, D), lambda i: (i, 0)),
            pl.BlockSpec((ROWB, 1), lambda i: (i, 0)),
            pl.BlockSpec((D, D), lambda i: (0, 0)),
            pl.BlockSpec((1, D), lambda i: (0, 0)),
        ],
        out_specs=pl.BlockSpec((ROWB, D), lambda i: (i, 0)),
        out_shape=jax.ShapeDtypeStruct((N, D), jnp.float32),
    )(eps.reshape(1), h, p0, p1, rdeg, w, b.reshape(1, D))


def kernel(features, edge_index, W0, b0, W1, b1, eps1, W2, b2, eps2, W3, b3, eps3):
    src = edge_index[0]
    dst = edge_index[1]
    pad = E_PAD - E
    ar = jnp.arange(pad, dtype=jnp.int32)
    src_p = jnp.concatenate([src, ar % N])
    dst_p = jnp.concatenate([dst, N + ar % (ACC_ROWS - N)])
    zeros_hbm = jnp.zeros((ROWS_PER_TILE, D), jnp.float32)
    ones_hbm = jnp.ones((CHUNK, D), jnp.float32)

    dp = _sc_degree(dst_p, zeros_hbm, ones_hbm)
    deg = dp[:N, 0] + dp[ACC_ROWS:ACC_ROWS + N, 0]
    rdeg = (1.0 / jnp.maximum(deg, 1.0)).reshape(N, 1)

    h = _tc_fc0(features, W0, b0)
    for w_, b_, eps_, relu in ((W1, b1, eps1, True),
                               (W2, b2, eps2, True),
                               (W3, b3, eps3, False)):
        p = _sc_segsum(h, src_p, dst_p, zeros_hbm)
        h = _tc_layer(h, p[:N], p[ACC_ROWS:ACC_ROWS + N], rdeg,
                      w_, b_, eps_, relu)
    return h
